# baseline (speedup 1.0000x reference)
import functools

import jax
import jax.numpy as jnp
from jax.experimental import pallas as pl
from jax.experimental.pallas import tpu as pltpu

_C = 256
_K = 8


def _stream_kernel(idx_ref, x_hbm, tbl_ref, out_hbm, in_buf, out_buf,
                   in_sems, out_sems, *, n_chunks):
    i0 = idx_ref[0]
    tbl = tbl_ref[...]
    rows = jax.lax.broadcasted_iota(jnp.int32, (tbl.shape[0], 1), 0)
    mask = (rows == i0).astype(tbl.dtype)
    row = jnp.sum(tbl * mask, axis=0, keepdims=True)

    def in_copy(c, k):
        return pltpu.make_async_copy(
            x_hbm.at[pl.ds(c * _C, _C), :], in_buf.at[k], in_sems.at[k])

    def out_copy(c, k):
        return pltpu.make_async_copy(
            out_buf.at[k], out_hbm.at[pl.ds(c * _C, _C), :], out_sems.at[k])

    for k in range(_K):
        in_copy(k, k).start()

    for i in range(n_chunks):
        k = i % _K
        in_copy(i, k).wait()
        if i >= _K:
            out_copy(i - _K, k).wait()
        out_buf[k] = in_buf[k] + row
        out_copy(i, k).start()
        if i + _K < n_chunks:
            in_copy(i + _K, k).start()

    for i in range(n_chunks - _K, n_chunks):
        out_copy(i, i % _K).wait()


def kernel(feats, table, modality_id):
    B, S, D = feats.shape
    N = B * S
    x = feats.reshape(N, D)
    n_rows = table.shape[0]
    idx = jnp.asarray(modality_id, jnp.int32).reshape(1)
    n_chunks = N // _C

    out = pl.pallas_call(
        functools.partial(_stream_kernel, n_chunks=n_chunks),
        grid_spec=pltpu.PrefetchScalarGridSpec(
            num_scalar_prefetch=1,
            grid=(1,),
            in_specs=[
                pl.BlockSpec(memory_space=pltpu.MemorySpace.HBM),
                pl.BlockSpec((n_rows, D), lambda i, idx_ref: (0, 0)),
            ],
            out_specs=pl.BlockSpec(memory_space=pltpu.MemorySpace.HBM),
            scratch_shapes=[
                pltpu.VMEM((_K, _C, D), jnp.float32),
                pltpu.VMEM((_K, _C, D), jnp.float32),
                pltpu.SemaphoreType.DMA((_K,)),
                pltpu.SemaphoreType.DMA((_K,)),
            ],
        ),
        out_shape=jax.ShapeDtypeStruct((N, D), feats.dtype),
    )(idx, x, table)
    return out.reshape(B, S, D)


# speedup vs baseline: 1.0083x; 1.0083x over previous
import functools

import jax
import jax.numpy as jnp
from jax.experimental import pallas as pl
from jax.experimental.pallas import tpu as pltpu

_C = 1024
_K = 4


def _stream_kernel(idx_ref, x_hbm, tbl_ref, out_hbm, in_buf, out_buf,
                   in_sems, out_sems, *, n_chunks):
    i0 = idx_ref[0]
    tbl = tbl_ref[...]
    rows = jax.lax.broadcasted_iota(jnp.int32, (tbl.shape[0], 1), 0)
    mask = (rows == i0).astype(tbl.dtype)
    row = jnp.sum(tbl * mask, axis=0, keepdims=True)

    def in_copy(c, k):
        return pltpu.make_async_copy(
            x_hbm.at[pl.ds(c * _C, _C), :], in_buf.at[k], in_sems.at[k])

    def out_copy(c, k):
        return pltpu.make_async_copy(
            out_buf.at[k], out_hbm.at[pl.ds(c * _C, _C), :], out_sems.at[k])

    for k in range(_K):
        in_copy(k, k).start()

    for i in range(n_chunks):
        k = i % _K
        in_copy(i, k).wait()
        if i >= _K:
            out_copy(i - _K, k).wait()
        out_buf[k] = in_buf[k] + row
        out_copy(i, k).start()
        if i + _K < n_chunks:
            in_copy(i + _K, k).start()

    for i in range(n_chunks - _K, n_chunks):
        out_copy(i, i % _K).wait()


def kernel(feats, table, modality_id):
    B, S, D = feats.shape
    N = B * S
    x = feats.reshape(N, D)
    n_rows = table.shape[0]
    idx = jnp.asarray(modality_id, jnp.int32).reshape(1)
    n_chunks = N // _C

    out = pl.pallas_call(
        functools.partial(_stream_kernel, n_chunks=n_chunks),
        grid_spec=pltpu.PrefetchScalarGridSpec(
            num_scalar_prefetch=1,
            grid=(1,),
            in_specs=[
                pl.BlockSpec(memory_space=pltpu.MemorySpace.HBM),
                pl.BlockSpec((n_rows, D), lambda i, idx_ref: (0, 0)),
            ],
            out_specs=pl.BlockSpec(memory_space=pltpu.MemorySpace.HBM),
            scratch_shapes=[
                pltpu.VMEM((_K, _C, D), jnp.float32),
                pltpu.VMEM((_K, _C, D), jnp.float32),
                pltpu.SemaphoreType.DMA((_K,)),
                pltpu.SemaphoreType.DMA((_K,)),
            ],
        ),
        out_shape=jax.ShapeDtypeStruct((N, D), feats.dtype),
    )(idx, x, table)
    return out.reshape(B, S, D)
